# baseline (speedup 1.0000x reference)
import functools

import jax
import jax.numpy as jnp
from jax import lax
from jax.experimental import pallas as pl
from jax.experimental.pallas import tpu as pltpu

_BB = 1


def _pool_one(g2, h2, wv, bv, *, n, k):
    f32 = jnp.float32
    bf16 = jnp.bfloat16

    wrep = jnp.broadcast_to(wv, (128, wv.shape[1]))
    w128 = lax.dot_general(h2, wrep, (((1,), (1,)), ((), ())),
                           preferred_element_type=f32) + bv
    s128 = jax.nn.sigmoid(w128)
    s_col = s128[:, 0:1]
    s_row = jnp.transpose(s128)[0:1, :]

    i_col = lax.broadcasted_iota(jnp.int32, (n, 1), 0).astype(f32)
    i_row = lax.broadcasted_iota(jnp.int32, (1, n), 1).astype(f32)
    beats = (s_row > s_col) | ((s_row == s_col) & (i_row < i_col))
    cmpf = beats.astype(f32)
    rank_col = jnp.sum(cmpf, axis=1, keepdims=True)
    rank128 = jnp.broadcast_to(rank_col, (n, 128))
    rank_row = jnp.transpose(rank128)[0:1, :]

    k_row = lax.broadcasted_iota(jnp.int32, (1, k), 1).astype(f32)
    k_col = lax.broadcasted_iota(jnp.int32, (k, 1), 0).astype(f32)
    pt = (rank_col == k_row).astype(f32)
    p = (rank_row == k_col).astype(f32)

    idx_row = jnp.sum(pt * i_col, axis=0, keepdims=True)
    vals = jnp.sum(p * s_row, axis=1, keepdims=True)

    dstd = (((1,), (0,)), ((), ()))
    hsel = lax.dot_general(p, h2, dstd, preferred_element_type=f32)
    new_h = hsel * vals

    gb = g2.astype(bf16)
    gr = lax.dot_general(p.astype(bf16), gb, dstd,
                         preferred_element_type=f32)
    gc = lax.dot_general(gb, pt.astype(bf16), dstd,
                         preferred_element_type=f32)
    m = lax.dot_general(gr.astype(bf16), gc.astype(bf16), dstd,
                        preferred_element_type=f32)
    th = (m > 0.5).astype(f32)
    deg = jnp.sum(th, axis=1, keepdims=True)
    return th / deg, new_h, idx_row.astype(jnp.int32)


def _pool_body(g_ref, h_ref, w_ref, b_ref, gnew_ref, newh_ref, idx_ref, *, n, k):
    wv = w_ref[...]
    bv = b_ref[0, 0]
    for bi in range(_BB):
        g_new, new_h, idx = _pool_one(g_ref[bi], h_ref[bi], wv, bv, n=n, k=k)
        gnew_ref[bi] = g_new
        newh_ref[bi] = new_h
        idx_ref[bi] = idx


def kernel(g, h, W, b):
    B, N, _ = g.shape
    D = h.shape[-1]
    K = max(2, int(0.5 * N))
    b2 = b.reshape(1, 1).astype(jnp.float32)

    out = pl.pallas_call(
        functools.partial(_pool_body, n=N, k=K),
        grid=(B // _BB,),
        in_specs=[
            pl.BlockSpec((_BB, N, N), lambda i: (i, 0, 0)),
            pl.BlockSpec((_BB, N, D), lambda i: (i, 0, 0)),
            pl.BlockSpec((1, D), lambda i: (0, 0)),
            pl.BlockSpec((1, 1), lambda i: (0, 0)),
        ],
        out_specs=[
            pl.BlockSpec((_BB, K, K), lambda i: (i, 0, 0)),
            pl.BlockSpec((_BB, K, D), lambda i: (i, 0, 0)),
            pl.BlockSpec((_BB, 1, K), lambda i: (i, 0, 0)),
        ],
        out_shape=[
            jax.ShapeDtypeStruct((B, K, K), jnp.float32),
            jax.ShapeDtypeStruct((B, K, D), jnp.float32),
            jax.ShapeDtypeStruct((B, 1, K), jnp.int32),
        ],
        compiler_params=pltpu.CompilerParams(
            dimension_semantics=("parallel",)),
    )(g, h, W, b2)
    g_new, new_h, idx = out
    return (g_new, new_h, idx.reshape(B, K))


# speedup vs baseline: 1.1038x; 1.1038x over previous
import functools

import jax
import jax.numpy as jnp
from jax import lax
from jax.experimental import pallas as pl
from jax.experimental.pallas import tpu as pltpu

_BB = 2


def _phase1(g2, h2, wv, bv, *, n, k):
    f32 = jnp.float32
    bf16 = jnp.bfloat16

    wrep = jnp.broadcast_to(wv, (128, wv.shape[1]))
    w128 = lax.dot_general(h2, wrep, (((1,), (1,)), ((), ())),
                           preferred_element_type=f32) + bv
    s128 = jax.nn.sigmoid(w128)
    s_col = s128[:, 0:1]
    s_row = jnp.transpose(s128)[0:1, :]

    i_col = lax.broadcasted_iota(jnp.int32, (n, 1), 0).astype(f32)
    i_row = lax.broadcasted_iota(jnp.int32, (1, n), 1).astype(f32)
    beats = (s_row > s_col) | ((s_row == s_col) & (i_row < i_col))
    cmpf = beats.astype(f32)
    rank_col = jnp.sum(cmpf, axis=1, keepdims=True)
    rank128 = jnp.broadcast_to(rank_col, (n, 128))
    rank_row = jnp.transpose(rank128)[0:1, :]

    k_row = lax.broadcasted_iota(jnp.int32, (1, k), 1).astype(f32)
    k_col = lax.broadcasted_iota(jnp.int32, (k, 1), 0).astype(f32)
    pt = (rank_col == k_row).astype(f32)
    p = (rank_row == k_col).astype(f32)

    idx_row = jnp.sum(pt * i_col, axis=0, keepdims=True)

    gb = g2.astype(bf16)
    hs = (h2 * s_col).astype(bf16)
    cat = jnp.concatenate([gb, hs], axis=1)
    return cat, p.astype(bf16), pt.astype(bf16), idx_row


def _phase2(cat, pb, ptb, idx_row, *, n, k):
    f32 = jnp.float32
    bf16 = jnp.bfloat16
    dstd = (((1,), (0,)), ((), ()))
    big = lax.dot_general(pb, cat, dstd,
                          preferred_element_type=f32)
    gr = big[:, :n].astype(bf16)
    new_h = big[:, n:]
    gc = lax.dot_general(cat[:, :n], ptb, dstd,
                         preferred_element_type=f32)
    m = lax.dot_general(gr, gc.astype(bf16), dstd,
                        preferred_element_type=f32)
    th = (m > 0.5).astype(f32)
    deg = jnp.sum(th, axis=1, keepdims=True)
    return th / deg, new_h, idx_row.astype(jnp.int32)


def _pool_body(g_ref, h_ref, w_ref, b_ref, gnew_ref, newh_ref, idx_ref, *, n, k):
    wv = w_ref[...]
    bv = b_ref[0, 0]
    mids = [_phase1(g_ref[bi], h_ref[bi], wv, bv, n=n, k=k)
            for bi in range(_BB)]
    for bi in range(_BB):
        g_new, new_h, idx = _phase2(*mids[bi], n=n, k=k)
        gnew_ref[bi] = g_new
        newh_ref[bi] = new_h
        idx_ref[bi] = idx


def kernel(g, h, W, b):
    B, N, _ = g.shape
    D = h.shape[-1]
    K = max(2, int(0.5 * N))
    b2 = b.reshape(1, 1).astype(jnp.float32)

    out = pl.pallas_call(
        functools.partial(_pool_body, n=N, k=K),
        grid=(B // _BB,),
        in_specs=[
            pl.BlockSpec((_BB, N, N), lambda i: (i, 0, 0)),
            pl.BlockSpec((_BB, N, D), lambda i: (i, 0, 0)),
            pl.BlockSpec((1, D), lambda i: (0, 0)),
            pl.BlockSpec((1, 1), lambda i: (0, 0)),
        ],
        out_specs=[
            pl.BlockSpec((_BB, K, K), lambda i: (i, 0, 0)),
            pl.BlockSpec((_BB, K, D), lambda i: (i, 0, 0)),
            pl.BlockSpec((_BB, 1, K), lambda i: (i, 0, 0)),
        ],
        out_shape=[
            jax.ShapeDtypeStruct((B, K, K), jnp.float32),
            jax.ShapeDtypeStruct((B, K, D), jnp.float32),
            jax.ShapeDtypeStruct((B, 1, K), jnp.int32),
        ],
        compiler_params=pltpu.CompilerParams(
            dimension_semantics=("parallel",)),
    )(g, h, W, b2)
    g_new, new_h, idx = out
    return (g_new, new_h, idx.reshape(B, K))


# speedup vs baseline: 1.1053x; 1.0014x over previous
import functools

import jax
import jax.numpy as jnp
from jax import lax
from jax.experimental import pallas as pl
from jax.experimental.pallas import tpu as pltpu

_BB = 2


def _phase1(g2, h2, wv, bv, *, n, k):
    f32 = jnp.float32
    bf16 = jnp.bfloat16

    wrep = jnp.broadcast_to(wv, (128, wv.shape[1]))
    w128 = lax.dot_general(h2, wrep, (((1,), (1,)), ((), ())),
                           preferred_element_type=f32) + bv
    s128 = jax.nn.sigmoid(w128)
    s_col = s128[:, 0:1]
    s_row = jnp.transpose(s128)[0:1, :]

    i_col = lax.broadcasted_iota(jnp.int32, (n, 1), 0).astype(f32)
    i_row = lax.broadcasted_iota(jnp.int32, (1, n), 1).astype(f32)
    beats = (s_row > s_col) | ((s_row == s_col) & (i_row < i_col))
    cmpf = beats.astype(f32)
    rank_col = jnp.sum(cmpf, axis=1, keepdims=True)
    rank128 = jnp.broadcast_to(rank_col, (n, 128))
    rank_row = jnp.transpose(rank128)[0:1, :]

    k_row = lax.broadcasted_iota(jnp.int32, (1, k), 1).astype(f32)
    k_col = lax.broadcasted_iota(jnp.int32, (k, 1), 0).astype(f32)
    pt = (rank_col == k_row).astype(f32)
    p = (rank_row == k_col).astype(f32)

    idx_row = jnp.sum(pt * i_col, axis=0, keepdims=True)

    gb = g2.astype(bf16)
    hs = (h2 * s_col).astype(bf16)
    cat = jnp.concatenate([gb, hs], axis=1)
    return cat, p.astype(bf16), pt.astype(bf16), idx_row


def _phase2(cat, pb, ptb, idx_row, *, n, k):
    f32 = jnp.float32
    bf16 = jnp.bfloat16
    dstd = (((1,), (0,)), ((), ()))
    big = lax.dot_general(pb, cat, dstd,
                          preferred_element_type=f32)
    gr = big[:, :n].astype(bf16)
    new_h = big[:, n:]
    gc = lax.dot_general(cat[:, :n], ptb, dstd,
                         preferred_element_type=f32)
    m = lax.dot_general(gr, gc.astype(bf16), dstd,
                        preferred_element_type=f32)
    th = (m > 0.5).astype(f32)
    deg = jnp.sum(th, axis=1, keepdims=True)
    return th / deg, new_h, idx_row.astype(jnp.int32)


def _pool_body(g_ref, h_ref, w_ref, b_ref, gnew_ref, newh_ref, idx_ref, *, n, k):
    wv = w_ref[...]
    bv = b_ref[0, 0]
    mids = [_phase1(g_ref[bi], h_ref[bi], wv, bv, n=n, k=k)
            for bi in range(_BB)]
    for bi in range(_BB):
        g_new, new_h, idx = _phase2(*mids[bi], n=n, k=k)
        gnew_ref[bi] = g_new
        newh_ref[bi] = new_h
        idx_ref[bi] = idx


def kernel(g, h, W, b):
    B, N, _ = g.shape
    D = h.shape[-1]
    K = max(2, int(0.5 * N))
    b2 = b.reshape(1, 1).astype(jnp.float32)

    out = pl.pallas_call(
        functools.partial(_pool_body, n=N, k=K),
        grid=(B // _BB,),
        in_specs=[
            pl.BlockSpec((_BB, N, N), lambda i: (i, 0, 0)),
            pl.BlockSpec((_BB, N, D), lambda i: (i, 0, 0)),
            pl.BlockSpec((1, D), lambda i: (0, 0)),
            pl.BlockSpec((1, 1), lambda i: (0, 0)),
        ],
        out_specs=[
            pl.BlockSpec((_BB, K, K), lambda i: (i, 0, 0)),
            pl.BlockSpec((_BB, K, D), lambda i: (i, 0, 0)),
            pl.BlockSpec((_BB, 1, K), lambda i: (i, 0, 0)),
        ],
        out_shape=[
            jax.ShapeDtypeStruct((B, K, K), jnp.float32),
            jax.ShapeDtypeStruct((B, K, D), jnp.float32),
            jax.ShapeDtypeStruct((B, 1, K), jnp.int32),
        ],
        compiler_params=pltpu.CompilerParams(
            dimension_semantics=("parallel",),
            vmem_limit_bytes=100 * 1024 * 1024),
    )(g, h, W, b2)
    g_new, new_h, idx = out
    return (g_new, new_h, idx.reshape(B, K))


# speedup vs baseline: 1.1119x; 1.0059x over previous
import functools

import jax
import jax.numpy as jnp
from jax import lax
from jax.experimental import pallas as pl
from jax.experimental.pallas import tpu as pltpu

_BB = 2


def _phase1(g2, h2, wv, bv, *, n, k):
    f32 = jnp.float32
    bf16 = jnp.bfloat16

    wrep = jnp.broadcast_to(wv, (8, wv.shape[1]))
    w128 = lax.dot_general(h2, wrep, (((1,), (1,)), ((), ())),
                           preferred_element_type=f32) + bv
    s128 = jax.nn.sigmoid(w128)
    s_col = s128[:, 0:1]
    s_row = jnp.transpose(s128)[0:1, :]

    i_col = lax.broadcasted_iota(jnp.int32, (n, 1), 0).astype(f32)
    i_row = lax.broadcasted_iota(jnp.int32, (1, n), 1).astype(f32)
    beats = (s_row > s_col) | ((s_row == s_col) & (i_row < i_col))
    cmpf = beats.astype(f32)
    rank_col = jnp.sum(cmpf, axis=1, keepdims=True)
    rank128 = jnp.broadcast_to(rank_col, (n, 8))
    rank_row = jnp.transpose(rank128)[0:1, :]

    k_row = lax.broadcasted_iota(jnp.int32, (1, k), 1).astype(f32)
    k_col = lax.broadcasted_iota(jnp.int32, (k, 1), 0).astype(f32)
    pt = (rank_col == k_row).astype(f32)
    p = (rank_row == k_col).astype(f32)

    idx_row = jnp.sum(pt * i_col, axis=0, keepdims=True)

    gb = g2.astype(bf16)
    hs = (h2 * s_col).astype(bf16)
    cat = jnp.concatenate([gb, hs], axis=1)
    return cat, p.astype(bf16), pt.astype(bf16), idx_row


def _phase2(cat, pb, ptb, idx_row, *, n, k):
    f32 = jnp.float32
    bf16 = jnp.bfloat16
    dstd = (((1,), (0,)), ((), ()))
    big = lax.dot_general(pb, cat, dstd,
                          preferred_element_type=f32)
    gr = big[:, :n].astype(bf16)
    new_h = big[:, n:]
    gc = lax.dot_general(cat[:, :n], ptb, dstd,
                         preferred_element_type=f32)
    m = lax.dot_general(gr, gc.astype(bf16), dstd,
                        preferred_element_type=f32)
    th = (m > 0.5).astype(f32)
    deg = jnp.sum(th, axis=1, keepdims=True)
    return th / deg, new_h, idx_row.astype(jnp.int32)


def _pool_body(g_ref, h_ref, w_ref, b_ref, gnew_ref, newh_ref, idx_ref, *, n, k):
    wv = w_ref[...]
    bv = b_ref[0, 0]
    mids = [_phase1(g_ref[bi], h_ref[bi], wv, bv, n=n, k=k)
            for bi in range(_BB)]
    for bi in range(_BB):
        g_new, new_h, idx = _phase2(*mids[bi], n=n, k=k)
        gnew_ref[bi] = g_new
        newh_ref[bi] = new_h
        idx_ref[bi] = idx


def kernel(g, h, W, b):
    B, N, _ = g.shape
    D = h.shape[-1]
    K = max(2, int(0.5 * N))
    b2 = b.reshape(1, 1).astype(jnp.float32)

    out = pl.pallas_call(
        functools.partial(_pool_body, n=N, k=K),
        grid=(B // _BB,),
        in_specs=[
            pl.BlockSpec((_BB, N, N), lambda i: (i, 0, 0)),
            pl.BlockSpec((_BB, N, D), lambda i: (i, 0, 0)),
            pl.BlockSpec((1, D), lambda i: (0, 0)),
            pl.BlockSpec((1, 1), lambda i: (0, 0)),
        ],
        out_specs=[
            pl.BlockSpec((_BB, K, K), lambda i: (i, 0, 0)),
            pl.BlockSpec((_BB, K, D), lambda i: (i, 0, 0)),
            pl.BlockSpec((_BB, 1, K), lambda i: (i, 0, 0)),
        ],
        out_shape=[
            jax.ShapeDtypeStruct((B, K, K), jnp.float32),
            jax.ShapeDtypeStruct((B, K, D), jnp.float32),
            jax.ShapeDtypeStruct((B, 1, K), jnp.int32),
        ],
        compiler_params=pltpu.CompilerParams(
            dimension_semantics=("parallel",),
            vmem_limit_bytes=100 * 1024 * 1024),
    )(g, h, W, b2)
    g_new, new_h, idx = out
    return (g_new, new_h, idx.reshape(B, K))
